# baseline (speedup 1.0000x reference)
import functools

import jax
import jax.numpy as jnp
from jax import lax
from jax.experimental import pallas as pl
from jax.experimental.pallas import tpu as pltpu
from jax.experimental.pallas import tpu_sc as plsc

N_NODES = 10000
NP = 10240
N_EDGES = 160000
EPAD = 163840
KK = 10
KL = 16
OP = 48
MW = 128
RW = 512
NC = 2
NS = 16
NW = NC * NS
EPW = EPAD // NW
CHUNK = 32
NCHUNK = EPW // CHUNK
GROWS = EPAD // 8
GPW = EPW // 8
ROWS_PER_TILE = NP // NS


def _gw_body(pk_ref, mp1_ref, mp2_ref, gw1_ref, gw2_ref):
    pk = pk_ref[...]
    p0 = pk[:, 0:1]
    p1 = pk[:, 1:2]

    def one(mp_ref):
        mu0 = mp_ref[0:1, :]
        mu1 = mp_ref[1:2, :]
        is0 = mp_ref[2:3, :]
        is1 = mp_ref[3:4, :]
        vmask = mp_ref[4:5, :]
        lane15 = mp_ref[5:6, :]
        d0 = (p0 - mu0) * is0
        d1 = (p1 - mu1) * is1
        g = jnp.exp(-0.5 * (d0 * d0 + d1 * d1))
        return g * vmask + lane15

    gw1_ref[...] = one(mp1_ref)
    gw2_ref[...] = one(mp2_ref)


def _edge_weights(pkor, mp1, mp2):
    be = 2000
    grid = N_EDGES // be
    return pl.pallas_call(
        _gw_body,
        grid=(grid,),
        in_specs=[
            pl.BlockSpec((be, 2), lambda i: (i, 0)),
            pl.BlockSpec((8, KL), lambda i: (0, 0)),
            pl.BlockSpec((8, KL), lambda i: (0, 0)),
        ],
        out_specs=[
            pl.BlockSpec((be, KL), lambda i: (i, 0)),
            pl.BlockSpec((be, KL), lambda i: (i, 0)),
        ],
        out_shape=[
            jax.ShapeDtypeStruct((N_EDGES, KL), jnp.float32),
            jax.ShapeDtypeStruct((N_EDGES, KL), jnp.float32),
        ],
    )(pkor, mp1, mp2)


def _mm_body(x_ref, w_ref, o_ref):
    o_ref[...] = jnp.dot(x_ref[...], w_ref[...],
                         preferred_element_type=jnp.float32)


def _project1(n_feat, w1p):
    bn = 1024
    return pl.pallas_call(
        _mm_body,
        grid=(NP // bn,),
        in_specs=[
            pl.BlockSpec((bn, 128), lambda i: (i, 0)),
            pl.BlockSpec((128, RW), lambda i: (0, 0)),
        ],
        out_specs=pl.BlockSpec((bn, RW), lambda i: (i, 0)),
        out_shape=jax.ShapeDtypeStruct((NP, RW), jnp.float32),
    )(n_feat, w1p)


def _layer2_body(part_ref, b1_ref, w2_ref, o_ref):
    agg = part_ref[0] + part_ref[1]
    deg = jnp.maximum(agg[:, 127:128], 1.0)
    h = agg[:, :40] / deg + b1_ref[0:1, :40]
    h = jnp.maximum(h, 0.0)
    o_ref[...] = jnp.dot(h, w2_ref[...], preferred_element_type=jnp.float32)


def _project2(part1, b1p, w2p):
    bn = 1024
    return pl.pallas_call(
        _layer2_body,
        grid=(NP // bn,),
        in_specs=[
            pl.BlockSpec((2, bn, MW), lambda i: (0, i, 0)),
            pl.BlockSpec((8, OP), lambda i: (0, 0)),
            pl.BlockSpec((40, RW), lambda i: (0, 0)),
        ],
        out_specs=pl.BlockSpec((bn, RW), lambda i: (i, 0)),
        out_shape=jax.ShapeDtypeStruct((NP, RW), jnp.float32),
    )(part1, b1p, w2p)


def _final_body(part_ref, b2_ref, o_ref):
    agg = part_ref[0] + part_ref[1]
    deg = jnp.maximum(agg[:, 127:128], 1.0)
    o_ref[...] = jnp.tanh(agg[:, :40] / deg + b2_ref[0:1, :40])


def _finalize(part2, b2p):
    bn = 1024
    return pl.pallas_call(
        _final_body,
        grid=(NP // bn,),
        in_specs=[
            pl.BlockSpec((2, bn, MW), lambda i: (0, i, 0)),
            pl.BlockSpec((8, OP), lambda i: (0, 0)),
        ],
        out_specs=pl.BlockSpec((bn, 40), lambda i: (i, 0)),
        out_shape=jax.ShapeDtypeStruct((NP, 40), jnp.float32),
    )(part2, b2p)


def _sc_edge_body(proj_hbm, src_hbm, dst_hbm, gwp_hbm, zero_hbm, part_hbm,
                  agg_sh, idx_v, dst_v, gwp_v, rows_v, msg_v,
                  sem_m, sem_w, sem_g0, sem_g1, sem_s):
    c = lax.axis_index("c")
    s = lax.axis_index("s")
    wid = s * NC + c

    zeros16 = jnp.zeros((16,), jnp.float32)
    lane = lax.iota(jnp.int32, 16)
    mask15 = jnp.where(lane == 15, 1.0, 0.0).astype(jnp.float32)

    pltpu.sync_copy(zero_hbm, agg_sh.at[pl.ds(s * ROWS_PER_TILE,
                                              ROWS_PER_TILE)])

    def mzrow(r, carry):
        for blk in range(3, 7):
            msg_v[r, pl.ds(blk * 16, 16)] = zeros16
        return carry

    lax.fori_loop(0, 2 * CHUNK, mzrow, 0)
    plsc.subcore_barrier()

    def fire_meta(i, m):
        base = wid * EPW + i * CHUNK
        pltpu.async_copy(src_hbm.at[pl.ds(base, CHUNK)],
                         idx_v.at[pl.ds(m * CHUNK, CHUNK)], sem_m)
        pltpu.async_copy(dst_hbm.at[pl.ds(base, CHUNK)],
                         dst_v.at[m], sem_m)

    def wait_meta(i, m):
        base = wid * EPW + i * CHUNK
        pltpu.make_async_copy(src_hbm.at[pl.ds(base, CHUNK)],
                              idx_v.at[pl.ds(m * CHUNK, CHUNK)],
                              sem_m).wait()
        pltpu.make_async_copy(dst_hbm.at[pl.ds(base, CHUNK)],
                              dst_v.at[m], sem_m).wait()

    def fire_gw(p, q):
        prow = wid * GPW + p * 8
        pltpu.async_copy(gwp_hbm.at[pl.ds(prow, 8), :],
                         gwp_v.at[pl.ds(q * 8, 8)], sem_w)

    def wait_gw(p, q):
        prow = wid * GPW + p * 8
        pltpu.make_async_copy(gwp_hbm.at[pl.ds(prow, 8), :],
                              gwp_v.at[pl.ds(q * 8, 8)], sem_w).wait()

    def fire_gather_dyn(m, b):
        @pl.when(b == 0)
        def _():
            pltpu.async_copy(proj_hbm.at[idx_v.at[pl.ds(m * CHUNK, CHUNK)]],
                             rows_v.at[pl.ds(0, CHUNK)], sem_g0)

        @pl.when(b == 1)
        def _():
            pltpu.async_copy(proj_hbm.at[idx_v.at[pl.ds(m * CHUNK, CHUNK)]],
                             rows_v.at[pl.ds(CHUNK, CHUNK)], sem_g1)

    def wait_gather_dyn(m, b):
        @pl.when(b == 0)
        def _():
            pltpu.make_async_copy(
                proj_hbm.at[idx_v.at[pl.ds(m * CHUNK, CHUNK)]],
                rows_v.at[pl.ds(0, CHUNK)], sem_g0).wait()

        @pl.when(b == 1)
        def _():
            pltpu.make_async_copy(
                proj_hbm.at[idx_v.at[pl.ds(m * CHUNK, CHUNK)]],
                rows_v.at[pl.ds(CHUNK, CHUNK)], sem_g1).wait()

    def wait_scatter(bm, mm):
        pltpu.make_async_copy(msg_v.at[pl.ds(bm * CHUNK, CHUNK)],
                              agg_sh.at[dst_v.at[mm]], sem_s).wait()

    fire_meta(0, 0)
    fire_meta(1, 1)
    fire_gw(0, 0)
    wait_meta(0, 0)
    fire_gather_dyn(0, 0)

    def chunk(i, carry):
        b = lax.rem(i, 2)
        b1 = lax.rem(i + 1, 2)
        m = lax.rem(i, 3)
        m1 = lax.rem(i + 1, 3)
        half = lax.rem(i, 2)
        p = lax.div(i, 2)
        q = lax.rem(p, 2)

        @pl.when(i + 1 < NCHUNK)
        def _():
            wait_meta(i + 1, m1)
            fire_gather_dyn(m1, b1)

        @pl.when((half == 0) & (i + 2 < NCHUNK))
        def _():
            fire_gw(p + 1, lax.rem(p + 1, 2))

        @pl.when(i >= 1)
        def _():
            wait_scatter(b1, lax.rem(i + 2, 3))

        @pl.when(i + 2 < NCHUNK)
        def _():
            fire_meta(i + 2, lax.rem(i + 2, 3))

        @pl.when(half == 0)
        def _():
            wait_gw(p, q)

        wait_gather_dyn(m, b)

        roff = b * CHUNK
        grow0 = q * 8 + half * 4

        def group(e8):
            gr = grow0 + e8
            for sub in range(8):
                e = e8 * 8 + sub
                gwv = gwp_v[gr, pl.ds(sub * 16, 16)]
                a0 = zeros16
                a1 = zeros16
                a2 = zeros16
                b0 = zeros16
                b1v = zeros16
                b2 = zeros16
                for k in range(0, KK, 2):
                    ga = gwv[k]
                    gb = gwv[k + 1]
                    a0 = a0 + ga * rows_v[roff + e, pl.ds(k * OP, 16)]
                    a1 = a1 + ga * rows_v[roff + e, pl.ds(k * OP + 16, 16)]
                    a2 = a2 + ga * rows_v[roff + e, pl.ds(k * OP + 32, 16)]
                    b0 = b0 + gb * rows_v[roff + e, pl.ds(k * OP + OP, 16)]
                    b1v = b1v + gb * rows_v[roff + e,
                                            pl.ds(k * OP + OP + 16, 16)]
                    b2 = b2 + gb * rows_v[roff + e,
                                          pl.ds(k * OP + OP + 32, 16)]
                msg_v[roff + e, pl.ds(0, 16)] = a0 + b0
                msg_v[roff + e, pl.ds(16, 16)] = a1 + b1v
                msg_v[roff + e, pl.ds(32, 16)] = a2 + b2
                msg_v[roff + e, pl.ds(112, 16)] = mask15 * gwv[15]

        for e8s in range(CHUNK // 8):
            group(e8s)
        pltpu.async_copy(msg_v.at[pl.ds(b * CHUNK, CHUNK)],
                         agg_sh.at[dst_v.at[m]], sem_s, add=True)
        return carry

    lax.fori_loop(0, NCHUNK, chunk, 0)
    wait_scatter((NCHUNK - 1) % 2, (NCHUNK - 1) % 3)
    plsc.subcore_barrier()

    pltpu.sync_copy(agg_sh.at[pl.ds(s * ROWS_PER_TILE, ROWS_PER_TILE)],
                    part_hbm.at[c, pl.ds(s * ROWS_PER_TILE, ROWS_PER_TILE)])


@functools.partial(
    pl.kernel,
    out_type=jax.ShapeDtypeStruct((NC, NP, MW), jnp.float32),
    mesh=plsc.VectorSubcoreMesh(core_axis_name="c", subcore_axis_name="s"),
    scratch_types=[
        pltpu.VMEM_SHARED((NP, MW), jnp.float32),
        pltpu.VMEM((3 * CHUNK,), jnp.int32),
        pltpu.VMEM((3, CHUNK), jnp.int32),
        pltpu.VMEM((16, MW), jnp.float32),
        pltpu.VMEM((2 * CHUNK, RW), jnp.float32),
        pltpu.VMEM((2 * CHUNK, MW), jnp.float32),
        pltpu.SemaphoreType.DMA,
        pltpu.SemaphoreType.DMA,
        pltpu.SemaphoreType.DMA,
        pltpu.SemaphoreType.DMA,
        pltpu.SemaphoreType.DMA,
    ],
)
def _sc_edge(proj_hbm, src_hbm, dst_hbm, gwp_hbm, zero_hbm, part_hbm,
             agg_sh, idx_v, dst_v, gwp_v, rows_v, msg_v, sem_m, sem_w,
             sem_g0, sem_g1, sem_s):
    _sc_edge_body(proj_hbm, src_hbm, dst_hbm, gwp_hbm, zero_hbm, part_hbm,
                  agg_sh, idx_v, dst_v, gwp_v, rows_v, msg_v,
                  sem_m, sem_w, sem_g0, sem_g1, sem_s)


def _pad_w(w):
    k, cin, cout = w.shape
    wp = jnp.pad(w, ((0, 0), (0, 0), (0, OP - cout)))
    flat = jnp.transpose(wp, (1, 0, 2)).reshape(cin, k * OP)
    return jnp.pad(flat, ((0, 0), (0, RW - k * OP)))


def _pack_mu(mu, inv_sigma):
    pad = KL - KK
    mu0 = jnp.pad(mu[:, 0], (0, pad))
    mu1 = jnp.pad(mu[:, 1], (0, pad))
    is0 = jnp.pad(inv_sigma[:, 0], (0, pad))
    is1 = jnp.pad(inv_sigma[:, 1], (0, pad))
    vmask = jnp.pad(jnp.ones((KK,), jnp.float32), (0, pad))
    lane15 = jnp.zeros((KL,), jnp.float32).at[15].set(1.0)
    zero = jnp.zeros((KL,), jnp.float32)
    return jnp.stack([mu0, mu1, is0, is1, vmask, lane15, zero, zero])


def _pad_bias(b):
    return jnp.pad(b, (0, OP - b.shape[0])).reshape(1, OP).repeat(8, 0)


def kernel(n_feat, edge_index, pkor, W1, mu1, inv_sigma1, b1,
           W2, mu2, inv_sigma2, b2):
    src = edge_index[0]
    dst = edge_index[1]
    n_feat = jnp.pad(n_feat, ((0, NP - N_NODES), (0, 0)))
    w1p = _pad_w(W1)
    w2p = _pad_w(W2)
    mp1 = _pack_mu(mu1, inv_sigma1)
    mp2 = _pack_mu(mu2, inv_sigma2)
    b1p = _pad_bias(b1)
    b2p = _pad_bias(b2)

    zero_tile = jnp.zeros((ROWS_PER_TILE, MW), jnp.float32)
    gw1, gw2 = _edge_weights(pkor, mp1, mp2)
    epad = EPAD - N_EDGES
    src = jnp.pad(src, (0, epad))
    dst = jnp.pad(dst, (0, epad))
    gw1 = jnp.pad(gw1, ((0, epad), (0, 0))).reshape(GROWS, 128)
    gw2 = jnp.pad(gw2, ((0, epad), (0, 0))).reshape(GROWS, 128)

    proj1 = _project1(n_feat, w1p)
    part1 = _sc_edge(proj1, src, dst, gw1, zero_tile)
    proj2 = _project2(part1, b1p, w2p)
    part2 = _sc_edge(proj2, src, dst, gw2, zero_tile)
    return _finalize(part2, b2p)[:N_NODES]


# speedup vs baseline: 1.0854x; 1.0854x over previous
import functools

import jax
import jax.numpy as jnp
from jax import lax
from jax.experimental import pallas as pl
from jax.experimental.pallas import tpu as pltpu
from jax.experimental.pallas import tpu_sc as plsc

N_NODES = 10000
NP = 10240
N_EDGES = 160000
EPAD = 163840
KK = 10
KL = 16
OP = 48
MW = 128
RW = 512
NC = 2
NS = 16
NW = NC * NS
EPW = EPAD // NW
CHUNK = 32
NCHUNK = EPW // CHUNK
GROWS = EPAD // 8
GPW = EPW // 8
ROWS_PER_TILE = NP // NS


def _gw_mm_body(pk_ref, mp1_ref, mp2_ref, x_ref, w_ref,
                gw1_ref, gw2_ref, o_ref):
    pk = pk_ref[...]
    p0 = pk[:, 0:1]
    p1 = pk[:, 1:2]
    i = pl.program_id(0)
    eidx = i * 2048 + jax.lax.broadcasted_iota(jnp.int32, (2048, 1), 0)
    valid = (eidx < N_EDGES).astype(jnp.float32)

    def one(mp_ref):
        mu0 = mp_ref[0:1, :]
        mu1 = mp_ref[1:2, :]
        is0 = mp_ref[2:3, :]
        is1 = mp_ref[3:4, :]
        vmask = mp_ref[4:5, :]
        lane15 = mp_ref[5:6, :]
        d0 = (p0 - mu0) * is0
        d1 = (p1 - mu1) * is1
        g = jnp.exp(-0.5 * (d0 * d0 + d1 * d1))
        return (g * vmask + lane15) * valid

    gw1_ref[...] = one(mp1_ref)
    gw2_ref[...] = one(mp2_ref)

    @pl.when(i < NP // 1024)
    def _():
        o_ref[...] = jnp.dot(x_ref[...], w_ref[...],
                             preferred_element_type=jnp.float32)


def _edge_weights_proj1(pkor_pad, mp1, mp2, n_feat, w1p):
    grid = EPAD // 2048
    return pl.pallas_call(
        _gw_mm_body,
        grid=(grid,),
        in_specs=[
            pl.BlockSpec((2048, 2), lambda i: (i, 0)),
            pl.BlockSpec((8, KL), lambda i: (0, 0)),
            pl.BlockSpec((8, KL), lambda i: (0, 0)),
            pl.BlockSpec((1024, 128), lambda i: (jnp.minimum(i, 9), 0)),
            pl.BlockSpec((128, RW), lambda i: (0, 0)),
        ],
        out_specs=[
            pl.BlockSpec((2048, KL), lambda i: (i, 0)),
            pl.BlockSpec((2048, KL), lambda i: (i, 0)),
            pl.BlockSpec((1024, RW), lambda i: (jnp.minimum(i, 9), 0)),
        ],
        out_shape=[
            jax.ShapeDtypeStruct((EPAD, KL), jnp.float32),
            jax.ShapeDtypeStruct((EPAD, KL), jnp.float32),
            jax.ShapeDtypeStruct((NP, RW), jnp.float32),
        ],
    )(pkor_pad, mp1, mp2, n_feat, w1p)


def _layer2_body(part_ref, b1_ref, w2_ref, o_ref):
    agg = part_ref[0] + part_ref[1]
    deg = jnp.maximum(agg[:, 127:128], 1.0)
    h = agg[:, :40] / deg + b1_ref[0:1, :40]
    h = jnp.maximum(h, 0.0)
    o_ref[...] = jnp.dot(h, w2_ref[...], preferred_element_type=jnp.float32)


def _project2(part1, b1p, w2p):
    bn = 1024
    return pl.pallas_call(
        _layer2_body,
        grid=(NP // bn,),
        in_specs=[
            pl.BlockSpec((2, bn, MW), lambda i: (0, i, 0)),
            pl.BlockSpec((8, OP), lambda i: (0, 0)),
            pl.BlockSpec((40, RW), lambda i: (0, 0)),
        ],
        out_specs=pl.BlockSpec((bn, RW), lambda i: (i, 0)),
        out_shape=jax.ShapeDtypeStruct((NP, RW), jnp.float32),
    )(part1, b1p, w2p)


def _final_body(part_ref, b2_ref, o_ref):
    agg = part_ref[0] + part_ref[1]
    deg = jnp.maximum(agg[:, 127:128], 1.0)
    o_ref[...] = jnp.tanh(agg[:, :40] / deg + b2_ref[0:1, :40])


def _finalize(part2, b2p):
    bn = 1024
    return pl.pallas_call(
        _final_body,
        grid=(NP // bn,),
        in_specs=[
            pl.BlockSpec((2, bn, MW), lambda i: (0, i, 0)),
            pl.BlockSpec((8, OP), lambda i: (0, 0)),
        ],
        out_specs=pl.BlockSpec((bn, 40), lambda i: (i, 0)),
        out_shape=jax.ShapeDtypeStruct((NP, 40), jnp.float32),
    )(part2, b2p)


def _sc_edge_body(proj_hbm, src_hbm, dst_hbm, gwp_hbm, zero_hbm, part_hbm,
                  agg_sh, idx_v, dst_v, gwp_v, rows_v, msg_v,
                  sem_m, sem_w, sem_g0, sem_g1, sem_s):
    c = lax.axis_index("c")
    s = lax.axis_index("s")
    wid = s * NC + c

    zeros16 = jnp.zeros((16,), jnp.float32)
    lane = lax.iota(jnp.int32, 16)
    mask15 = jnp.where(lane == 15, 1.0, 0.0).astype(jnp.float32)

    pltpu.sync_copy(zero_hbm, agg_sh.at[pl.ds(s * ROWS_PER_TILE,
                                              ROWS_PER_TILE)])

    def mzrow(r, carry):
        for blk in range(3, 7):
            msg_v[r, pl.ds(blk * 16, 16)] = zeros16
        return carry

    lax.fori_loop(0, 2 * CHUNK, mzrow, 0)
    plsc.subcore_barrier()

    def fire_meta(i, m):
        base = wid * EPW + i * CHUNK
        pltpu.async_copy(src_hbm.at[pl.ds(base, CHUNK)],
                         idx_v.at[pl.ds(m * CHUNK, CHUNK)], sem_m)
        pltpu.async_copy(dst_hbm.at[pl.ds(base, CHUNK)],
                         dst_v.at[m], sem_m)

    def wait_meta(i, m):
        base = wid * EPW + i * CHUNK
        pltpu.make_async_copy(src_hbm.at[pl.ds(base, CHUNK)],
                              idx_v.at[pl.ds(m * CHUNK, CHUNK)],
                              sem_m).wait()
        pltpu.make_async_copy(dst_hbm.at[pl.ds(base, CHUNK)],
                              dst_v.at[m], sem_m).wait()

    def fire_gw(p, q):
        prow = wid * GPW + p * 8
        pltpu.async_copy(gwp_hbm.at[pl.ds(prow, 8), :],
                         gwp_v.at[pl.ds(q * 8, 8)], sem_w)

    def wait_gw(p, q):
        prow = wid * GPW + p * 8
        pltpu.make_async_copy(gwp_hbm.at[pl.ds(prow, 8), :],
                              gwp_v.at[pl.ds(q * 8, 8)], sem_w).wait()

    def fire_gather_dyn(m, b):
        @pl.when(b == 0)
        def _():
            pltpu.async_copy(proj_hbm.at[idx_v.at[pl.ds(m * CHUNK, CHUNK)]],
                             rows_v.at[pl.ds(0, CHUNK)], sem_g0)

        @pl.when(b == 1)
        def _():
            pltpu.async_copy(proj_hbm.at[idx_v.at[pl.ds(m * CHUNK, CHUNK)]],
                             rows_v.at[pl.ds(CHUNK, CHUNK)], sem_g1)

    def wait_gather_dyn(m, b):
        @pl.when(b == 0)
        def _():
            pltpu.make_async_copy(
                proj_hbm.at[idx_v.at[pl.ds(m * CHUNK, CHUNK)]],
                rows_v.at[pl.ds(0, CHUNK)], sem_g0).wait()

        @pl.when(b == 1)
        def _():
            pltpu.make_async_copy(
                proj_hbm.at[idx_v.at[pl.ds(m * CHUNK, CHUNK)]],
                rows_v.at[pl.ds(CHUNK, CHUNK)], sem_g1).wait()

    def wait_scatter(bm, mm):
        pltpu.make_async_copy(msg_v.at[pl.ds(bm * CHUNK, CHUNK)],
                              agg_sh.at[dst_v.at[mm]], sem_s).wait()

    fire_meta(0, 0)
    fire_meta(1, 1)
    fire_gw(0, 0)
    wait_meta(0, 0)
    fire_gather_dyn(0, 0)

    def chunk(i, carry):
        b = lax.rem(i, 2)
        b1 = lax.rem(i + 1, 2)
        m = lax.rem(i, 3)
        m1 = lax.rem(i + 1, 3)
        half = lax.rem(i, 2)
        p = lax.div(i, 2)
        q = lax.rem(p, 2)

        @pl.when(i + 1 < NCHUNK)
        def _():
            wait_meta(i + 1, m1)
            fire_gather_dyn(m1, b1)

        @pl.when((half == 0) & (i + 2 < NCHUNK))
        def _():
            fire_gw(p + 1, lax.rem(p + 1, 2))

        @pl.when(half == 0)
        def _():
            wait_gw(p, q)

        wait_gather_dyn(m, b)

        @pl.when(i >= 1)
        def _():
            wait_scatter(b1, lax.rem(i + 2, 3))

        @pl.when(i + 2 < NCHUNK)
        def _():
            fire_meta(i + 2, lax.rem(i + 2, 3))

        roff = b * CHUNK
        grow0 = q * 8 + half * 4

        def group(e8, ecarry):
            gr = grow0 + e8
            for sub in range(8):
                e = e8 * 8 + sub
                gwv = gwp_v[gr, pl.ds(sub * 16, 16)]
                a0 = zeros16
                a1 = zeros16
                a2 = zeros16
                b0 = zeros16
                b1v = zeros16
                b2 = zeros16
                for k in range(0, KK, 2):
                    ga = gwv[k]
                    gb = gwv[k + 1]
                    a0 = a0 + ga * rows_v[roff + e, pl.ds(k * OP, 16)]
                    a1 = a1 + ga * rows_v[roff + e, pl.ds(k * OP + 16, 16)]
                    a2 = a2 + ga * rows_v[roff + e, pl.ds(k * OP + 32, 16)]
                    b0 = b0 + gb * rows_v[roff + e, pl.ds(k * OP + OP, 16)]
                    b1v = b1v + gb * rows_v[roff + e,
                                            pl.ds(k * OP + OP + 16, 16)]
                    b2 = b2 + gb * rows_v[roff + e,
                                          pl.ds(k * OP + OP + 32, 16)]
                msg_v[roff + e, pl.ds(0, 16)] = a0 + b0
                msg_v[roff + e, pl.ds(16, 16)] = a1 + b1v
                msg_v[roff + e, pl.ds(32, 16)] = a2 + b2
                msg_v[roff + e, pl.ds(112, 16)] = mask15 * gwv[15]
            return ecarry

        lax.fori_loop(0, CHUNK // 8, group, 0)
        pltpu.async_copy(msg_v.at[pl.ds(b * CHUNK, CHUNK)],
                         agg_sh.at[dst_v.at[m]], sem_s, add=True)
        return carry

    lax.fori_loop(0, NCHUNK, chunk, 0)
    wait_scatter((NCHUNK - 1) % 2, (NCHUNK - 1) % 3)
    plsc.subcore_barrier()

    pltpu.sync_copy(agg_sh.at[pl.ds(s * ROWS_PER_TILE, ROWS_PER_TILE)],
                    part_hbm.at[c, pl.ds(s * ROWS_PER_TILE, ROWS_PER_TILE)])


@functools.partial(
    pl.kernel,
    out_type=jax.ShapeDtypeStruct((NC, NP, MW), jnp.float32),
    mesh=plsc.VectorSubcoreMesh(core_axis_name="c", subcore_axis_name="s"),
    scratch_types=[
        pltpu.VMEM_SHARED((NP, MW), jnp.float32),
        pltpu.VMEM((3 * CHUNK,), jnp.int32),
        pltpu.VMEM((3, CHUNK), jnp.int32),
        pltpu.VMEM((16, MW), jnp.float32),
        pltpu.VMEM((2 * CHUNK, RW), jnp.float32),
        pltpu.VMEM((2 * CHUNK, MW), jnp.float32),
        pltpu.SemaphoreType.DMA,
        pltpu.SemaphoreType.DMA,
        pltpu.SemaphoreType.DMA,
        pltpu.SemaphoreType.DMA,
        pltpu.SemaphoreType.DMA,
    ],
)
def _sc_edge(proj_hbm, src_hbm, dst_hbm, gwp_hbm, zero_hbm, part_hbm,
             agg_sh, idx_v, dst_v, gwp_v, rows_v, msg_v, sem_m, sem_w,
             sem_g0, sem_g1, sem_s):
    _sc_edge_body(proj_hbm, src_hbm, dst_hbm, gwp_hbm, zero_hbm, part_hbm,
                  agg_sh, idx_v, dst_v, gwp_v, rows_v, msg_v,
                  sem_m, sem_w, sem_g0, sem_g1, sem_s)


def _pad_w(w):
    k, cin, cout = w.shape
    wp = jnp.pad(w, ((0, 0), (0, 0), (0, OP - cout)))
    flat = jnp.transpose(wp, (1, 0, 2)).reshape(cin, k * OP)
    return jnp.pad(flat, ((0, 0), (0, RW - k * OP)))


def _pack_mu(mu, inv_sigma):
    pad = KL - KK
    mu0 = jnp.pad(mu[:, 0], (0, pad))
    mu1 = jnp.pad(mu[:, 1], (0, pad))
    is0 = jnp.pad(inv_sigma[:, 0], (0, pad))
    is1 = jnp.pad(inv_sigma[:, 1], (0, pad))
    vmask = jnp.pad(jnp.ones((KK,), jnp.float32), (0, pad))
    lane15 = jnp.zeros((KL,), jnp.float32).at[15].set(1.0)
    zero = jnp.zeros((KL,), jnp.float32)
    return jnp.stack([mu0, mu1, is0, is1, vmask, lane15, zero, zero])


def _pad_bias(b):
    return jnp.pad(b, (0, OP - b.shape[0])).reshape(1, OP).repeat(8, 0)


def kernel(n_feat, edge_index, pkor, W1, mu1, inv_sigma1, b1,
           W2, mu2, inv_sigma2, b2):
    src = edge_index[0]
    dst = edge_index[1]
    n_feat = jnp.pad(n_feat, ((0, NP - N_NODES), (0, 0)))
    w1p = _pad_w(W1)
    w2p = _pad_w(W2)
    mp1 = _pack_mu(mu1, inv_sigma1)
    mp2 = _pack_mu(mu2, inv_sigma2)
    b1p = _pad_bias(b1)
    b2p = _pad_bias(b2)

    zero_tile = jnp.zeros((ROWS_PER_TILE, MW), jnp.float32)
    epad = EPAD - N_EDGES
    src = jnp.pad(src, (0, epad))
    dst = jnp.pad(dst, (0, epad))
    pkor_pad = jnp.pad(pkor, ((0, epad), (0, 0)))

    gw1, gw2, proj1 = _edge_weights_proj1(pkor_pad, mp1, mp2, n_feat, w1p)
    gw1 = gw1.reshape(GROWS, 128)
    gw2 = gw2.reshape(GROWS, 128)
    part1 = _sc_edge(proj1, src, dst, gw1, zero_tile)
    proj2 = _project2(part1, b1p, w2p)
    part2 = _sc_edge(proj2, src, dst, gw2, zero_tile)
    return _finalize(part2, b2p)[:N_NODES]
